# baseline (speedup 1.0000x reference)
import functools

import jax
import jax.numpy as jnp
from jax import lax
from jax.experimental import pallas as pl
from jax.experimental.pallas import tpu as pltpu
from jax.experimental.pallas import tpu_sc as plsc

_K = 16
_EPS = 1e-5
_QC = 32
_HI = lax.Precision.HIGHEST


def _k0_body(f2t, p2t, w1f, w1p, a_o):
    a_o[0] = (jnp.dot(f2t[0], w1f[...], preferred_element_type=jnp.float32)
              + jnp.dot(p2t[0], w1p[...], preferred_element_type=jnp.float32))


def _k1_body(p1t, p2, w1p, idx_o, c_o, sums_o, *, blk, n2):
    q = p1t[0]
    s3 = p2[0]

    cblk = jnp.dot(q, w1p[...], preferred_element_type=jnp.float32)

    qq = jnp.sum(q * q, axis=1, keepdims=True)
    ss = jnp.sum(s3 * s3, axis=0, keepdims=True)
    qs = jnp.dot(q, s3, preferred_element_type=jnp.float32)
    d = (qq - 2.0 * qs) + ss

    iota = lax.broadcasted_iota(jnp.int32, (blk, n2), 1)
    iota_k = lax.broadcasted_iota(jnp.int32, (blk, _K), 1)
    inf = jnp.float32(jnp.inf)
    boff = pl.program_id(0) * n2

    acc = jnp.zeros((blk, _K), jnp.int32)
    for t in range(_K):
        rmin = jnp.min(d, axis=1, keepdims=True)
        idx_t = jnp.min(jnp.where(d == rmin, iota, n2), axis=1, keepdims=True)
        d = jnp.where(iota == idx_t, inf, d)
        acc = jnp.where(iota_k == t, idx_t + boff, acc)

    idx_o[0] = acc
    c_o[0] = cblk
    c64 = cblk[:, :64]
    part = jnp.concatenate([
        jnp.sum(c64, axis=0, keepdims=True),
        jnp.sum(c64 * c64, axis=0, keepdims=True),
        jnp.zeros((6, 64), jnp.float32),
    ], axis=0)

    @pl.when((pl.program_id(0) == 0) & (pl.program_id(1) == 0))
    def _():
        sums_o[...] = jnp.zeros_like(sums_o)

    sums_o[...] += part


def _sc_gather(a_hbm, idx_hbm, c_hbm, mmn_hbm, p_hbm,
               idx_v, rows_v, c_v, mmn_v, p_v, sem, *, qpw, nc):
    wid = lax.axis_index("s") * nc + lax.axis_index("c")
    zero = jnp.zeros((16,), jnp.float32)
    nsub = (_QC * _K) // 128

    nchunks = qpw // _QC

    def chunk(ci, carry):
        qbase = wid * qpw + ci * _QC
        pltpu.sync_copy(idx_hbm.at[wid * nchunks + ci], idx_v)
        copies = [pltpu.async_copy(a_hbm.at[idx_v.at[i]],
                                   rows_v.at[pl.ds(i * 128, 128)], sem)
                  for i in range(nsub)]
        for cp in copies:
            cp.wait()
        pltpu.sync_copy(c_hbm.at[pl.ds(qbase, _QC)], c_v)

        def qloop(qi, pc):
            out = list(pc)
            for cb in range(4):
                sl = slice(16 * cb, 16 * (cb + 1))
                m = jnp.full((16,), -jnp.inf, jnp.float32)
                mn = jnp.full((16,), jnp.inf, jnp.float32)
                s = zero
                qsum = zero
                for j in range(_K):
                    f = rows_v[qi * _K + j, sl]
                    m = jnp.maximum(m, f)
                    mn = jnp.minimum(mn, f)
                    s = s + f
                    qsum = qsum + f * f
                mmn_v[qi, sl] = m
                mmn_v[qi, slice(64 + 16 * cb, 64 + 16 * (cb + 1))] = mn
                cv = c_v[qi, sl]
                out[cb] = out[cb] + s
                out[4 + cb] = out[4 + cb] + qsum
                out[8 + cb] = out[8 + cb] + cv * s
            return tuple(out)

        pc = lax.fori_loop(0, _QC, qloop, carry)
        pltpu.sync_copy(mmn_v, mmn_hbm.at[pl.ds(qbase, _QC)])
        return pc

    pc = lax.fori_loop(0, nchunks, chunk, tuple(zero for _ in range(12)))
    for r in range(16):
        for l in range(8):
            p_v[r, slice(16 * l, 16 * (l + 1))] = pc[r] if (r < 12 and l == 0) else zero
    pltpu.sync_copy(p_v, p_hbm.at[wid])


def _bn1_consts(sums, p_red, g1r, b1r, n1_total):
    sum_s, sum_q, sum_cs = p_red[0:1], p_red[1:2], p_red[2:3]
    sum_c, sum_c2 = sums[0:1], sums[1:2]
    mean1 = (sum_s - _K * sum_c) / n1_total
    ey2 = (sum_q - 2.0 * sum_cs + _K * sum_c2) / n1_total
    var1 = ey2 - mean1 * mean1
    alpha = g1r[...] * lax.rsqrt(var1 + _EPS)
    beta = b1r[...] - alpha * mean1
    return alpha, beta


def _reduce_partials(p_ref):
    return jnp.sum(p_ref[...], axis=0)


def _stage1(mmn_ref, c_ref, f1t_ref, alpha, beta):
    mmn = mmn_ref[0]
    msel = jnp.where(alpha >= 0.0, mmn[:, :64], mmn[:, 64:])
    h = jnp.maximum(alpha * (msel - c_ref[0][:, :64]) + beta, 0.0)
    return jnp.concatenate([h, f1t_ref[0]], axis=1)


def _k2_body(mmn_i, c_i, f1t_i, sums_i, p_i, g1r, b1r, g_o, sz_o,
             *, n1_total):
    alpha, beta = _bn1_consts(sums_i[...], _reduce_partials(p_i), g1r, b1r,
                              n1_total)
    z = _stage1(mmn_i, c_i, f1t_i, alpha, beta)
    ztz = lax.dot_general(z, z, (((0,), (0,)), ((), ())),
                          preferred_element_type=jnp.float32, precision=_HI)
    szrow = jnp.sum(z, axis=0, keepdims=True)

    @pl.when((pl.program_id(0) == 0) & (pl.program_id(1) == 0))
    def _():
        g_o[...] = jnp.zeros_like(g_o)
        sz_o[...] = jnp.zeros_like(sz_o)

    g_o[...] += ztz
    sz_o[...] += jnp.concatenate([szrow, jnp.zeros((7, 128), jnp.float32)], 0)


def _k3_body(mmn_i, c_i, f1t_i, sums_i, p_i, g1r, b1r, g_i, sz_i, w2t,
             g2r, b2r, y_o, *, n1_total, n2_total):
    alpha, beta = _bn1_consts(sums_i[...], _reduce_partials(p_i), g1r, b1r,
                              n1_total)
    z = _stage1(mmn_i, c_i, f1t_i, alpha, beta)

    szrow = sz_i[0:1] / n2_total
    m2 = jnp.dot(szrow, w2t[...], preferred_element_type=jnp.float32,
                 precision=_HI)
    t = jnp.dot(g_i[...], w2t[...], preferred_element_type=jnp.float32,
                precision=_HI)
    e2 = jnp.sum(w2t[...] * t, axis=0, keepdims=True) / n2_total
    inv2 = lax.rsqrt((e2 - m2 * m2) + _EPS)
    scale = g2r[...] * inv2
    bias = b2r[...] - m2 * scale

    y = jnp.dot(z, w2t[...], preferred_element_type=jnp.float32)
    y_o[0] = jnp.maximum(y * scale + bias, 0.0)


def kernel(pos1, pos2, feature1, feature2, W1, g1, b1, W2, g2, b2):
    B, _, N1 = pos1.shape
    N2 = pos2.shape[2]
    f32 = jnp.float32

    pos1t = jnp.transpose(pos1, (0, 2, 1))
    pos2t = jnp.transpose(pos2, (0, 2, 1))
    f2t = jnp.transpose(feature2, (0, 2, 1))
    f1t = jnp.transpose(feature1, (0, 2, 1))
    w1t = jnp.transpose(W1)
    w1f = jnp.pad(w1t[:64], ((0, 0), (0, 64)))
    w1p = jnp.pad(w1t[64:], ((0, 0), (0, 64)))
    w2t = jnp.transpose(W2)
    g1r, b1r = g1.reshape(1, 64), b1.reshape(1, 64)
    g2r, b2r = g2.reshape(1, 128), b2.reshape(1, 128)

    whole = lambda shp: pl.BlockSpec(shp, lambda b, i: (0,) * len(shp))

    a_tab = pl.pallas_call(
        _k0_body,
        grid=(B,),
        in_specs=[pl.BlockSpec((1, N2, 64), lambda b: (b, 0, 0)),
                  pl.BlockSpec((1, N2, 3), lambda b: (b, 0, 0)),
                  pl.BlockSpec((64, 128), lambda b: (0, 0)),
                  pl.BlockSpec((3, 128), lambda b: (0, 0))],
        out_specs=[pl.BlockSpec((1, N2, 128), lambda b: (b, 0, 0))],
        out_shape=[jax.ShapeDtypeStruct((B, N2, 128), f32)],
    )(f2t, pos2t, w1f, w1p)[0]

    blk = 1024
    grid = (B, N1 // blk)
    rowblk = lambda w: pl.BlockSpec((1, blk, w), lambda b, i: (b, i, 0))

    idx_arr, c_arr, sums = pl.pallas_call(
        functools.partial(_k1_body, blk=blk, n2=N2),
        grid=grid,
        in_specs=[rowblk(3), pl.BlockSpec((1, 3, N2), lambda b, i: (b, 0, 0)),
                  whole((3, 128))],
        out_specs=[rowblk(_K), rowblk(128), whole((8, 64))],
        out_shape=[jax.ShapeDtypeStruct((B, N1, _K), jnp.int32),
                   jax.ShapeDtypeStruct((B, N1, 128), f32),
                   jax.ShapeDtypeStruct((8, 64), f32)],
    )(pos1t, pos2, w1p)

    info = plsc.get_sparse_core_info()
    nc, ns = info.num_cores, info.num_subcores
    nw = nc * ns
    bn1 = B * N1
    qpw = bn1 // nw
    a_pad = a_tab.reshape(B * N2, 128)
    idx_3d = idx_arr.reshape(bn1 // _QC, (_QC * _K) // 128, 128)
    c_pad = c_arr.reshape(bn1, 128)

    sc = functools.partial(
        pl.kernel,
        mesh=plsc.VectorSubcoreMesh(core_axis_name="c", subcore_axis_name="s"),
        out_type=[jax.ShapeDtypeStruct((bn1, 128), f32),
                  jax.ShapeDtypeStruct((nw, 16, 128), f32)],
        scratch_types=[pltpu.VMEM(((_QC * _K) // 128, 128), jnp.int32),
                       pltpu.VMEM((_QC * _K, 128), f32),
                       pltpu.VMEM((_QC, 128), f32),
                       pltpu.VMEM((_QC, 128), f32),
                       pltpu.VMEM((16, 128), f32),
                       pltpu.SemaphoreType.DMA],
    )(functools.partial(_sc_gather, qpw=qpw, nc=nc))
    mmn_flat, p_arr = sc(a_pad, idx_3d, c_pad)

    mmn_arr = mmn_flat.reshape(B, N1, 128)
    p_in = p_arr[:, :12, :16].reshape(nw, 3, 64)
    p_in = jnp.concatenate([p_in, jnp.zeros((nw, 5, 64), f32)], axis=1)

    n1_total = float(B * N1 * _K)
    n2_total = float(B * N1)

    blk2 = 512
    grid2 = (B, N1 // blk2)
    rowblk2 = lambda w: pl.BlockSpec((1, blk2, w), lambda b, i: (b, i, 0))
    whole2 = lambda shp: pl.BlockSpec(shp, lambda b, i: (0,) * len(shp))

    g_mat, sz = pl.pallas_call(
        functools.partial(_k2_body, n1_total=n1_total),
        grid=grid2,
        in_specs=[rowblk2(128), rowblk2(128), rowblk2(64),
                  whole2((8, 64)), whole2((nw, 8, 64)),
                  whole2((1, 64)), whole2((1, 64))],
        out_specs=[whole2((128, 128)), whole2((8, 128))],
        out_shape=[jax.ShapeDtypeStruct((128, 128), f32),
                   jax.ShapeDtypeStruct((8, 128), f32)],
    )(mmn_arr, c_arr, f1t, sums, p_in, g1r, b1r)

    y = pl.pallas_call(
        functools.partial(_k3_body, n1_total=n1_total, n2_total=n2_total),
        grid=grid2,
        in_specs=[rowblk2(128), rowblk2(128), rowblk2(64),
                  whole2((8, 64)), whole2((nw, 8, 64)),
                  whole2((1, 64)), whole2((1, 64)),
                  whole2((128, 128)), whole2((8, 128)), whole2((128, 128)),
                  whole2((1, 128)), whole2((1, 128))],
        out_specs=[rowblk2(128)],
        out_shape=[jax.ShapeDtypeStruct((B, N1, 128), f32)],
    )(mmn_arr, c_arr, f1t, sums, p_in, g1r, b1r, g_mat, sz, w2t,
      g2r, b2r)[0]

    return jnp.transpose(y, (0, 2, 1))


# speedup vs baseline: 1.0519x; 1.0519x over previous
import functools

import jax
import jax.numpy as jnp
from jax import lax
from jax.experimental import pallas as pl
from jax.experimental.pallas import tpu as pltpu
from jax.experimental.pallas import tpu_sc as plsc

_K = 16
_EPS = 1e-5
_QC = 32
_HI = lax.Precision.HIGHEST


def _k0_body(f2t, p2t, w1f, w1p, a_o):
    a_o[0] = (jnp.dot(f2t[0], w1f[...], preferred_element_type=jnp.float32)
              + jnp.dot(p2t[0], w1p[...], preferred_element_type=jnp.float32))


def _k1_body(p1t, p2, w1p, idx_o, c_o, sums_o, *, blk, n2):
    q = p1t[0]
    s3 = p2[0]

    cblk = jnp.dot(q, w1p[...], preferred_element_type=jnp.float32)

    qq = jnp.sum(q * q, axis=1, keepdims=True)
    ss = jnp.sum(s3 * s3, axis=0, keepdims=True)
    qs = jnp.dot(q, s3, preferred_element_type=jnp.float32)
    d = (qq - 2.0 * qs) + ss

    iota = lax.broadcasted_iota(jnp.int32, (blk, n2), 1)
    iota_k = lax.broadcasted_iota(jnp.int32, (blk, _K), 1)
    inf = jnp.float32(jnp.inf)
    boff = pl.program_id(0) * n2

    acc = jnp.zeros((blk, _K), jnp.int32)
    for t in range(_K):
        rmin = jnp.min(d, axis=1, keepdims=True)
        idx_t = jnp.min(jnp.where(d == rmin, iota, n2), axis=1, keepdims=True)
        d = jnp.where(iota == idx_t, inf, d)
        acc = jnp.where(iota_k == t, idx_t + boff, acc)

    idx_o[0] = acc
    c_o[0] = cblk
    c64 = cblk[:, :64]
    part = jnp.concatenate([
        jnp.sum(c64, axis=0, keepdims=True),
        jnp.sum(c64 * c64, axis=0, keepdims=True),
        jnp.zeros((6, 64), jnp.float32),
    ], axis=0)

    @pl.when((pl.program_id(0) == 0) & (pl.program_id(1) == 0))
    def _():
        sums_o[...] = jnp.zeros_like(sums_o)

    sums_o[...] += part


def _sc_gather(a_hbm, idx_hbm, c_hbm, mmn_hbm, p_hbm,
               idx_v, rows_v, c_v, mmn_v, p_v, sem, *, qpw, nc):
    wid = lax.axis_index("s") * nc + lax.axis_index("c")
    zero = jnp.zeros((16,), jnp.float32)
    nsub = (_QC * _K) // 128

    nchunks = qpw // _QC

    def chunk(ci, carry):
        qbase = wid * qpw + ci * _QC
        pltpu.sync_copy(idx_hbm.at[wid * nchunks + ci], idx_v)
        copies = [pltpu.async_copy(a_hbm.at[idx_v.at[i]],
                                   rows_v.at[pl.ds(i * 128, 128)], sem)
                  for i in range(nsub)]
        for cp in copies:
            cp.wait()
        pltpu.sync_copy(c_hbm.at[pl.ds(qbase, _QC)], c_v)

        def qloop(qi, pc):
            out = list(pc)
            for cb in range(4):
                sl = slice(16 * cb, 16 * (cb + 1))
                m = jnp.full((16,), -jnp.inf, jnp.float32)
                mn = jnp.full((16,), jnp.inf, jnp.float32)
                s = zero
                qsum = zero
                for j in range(_K):
                    f = rows_v[qi * _K + j, sl]
                    m = jnp.maximum(m, f)
                    mn = jnp.minimum(mn, f)
                    s = s + f
                    qsum = qsum + f * f
                mmn_v[qi, sl] = m
                mmn_v[qi, slice(64 + 16 * cb, 64 + 16 * (cb + 1))] = mn
                cv = c_v[qi, sl]
                out[cb] = out[cb] + s
                out[4 + cb] = out[4 + cb] + qsum
                out[8 + cb] = out[8 + cb] + cv * s
            return tuple(out)

        pc = lax.fori_loop(0, _QC, qloop, carry)
        pltpu.sync_copy(mmn_v, mmn_hbm.at[pl.ds(qbase, _QC)])
        return pc

    pc = lax.fori_loop(0, nchunks, chunk, tuple(zero for _ in range(12)))
    for r in range(16):
        for l in range(8):
            p_v[r, slice(16 * l, 16 * (l + 1))] = pc[r] if (r < 12 and l == 0) else zero
    pltpu.sync_copy(p_v, p_hbm.at[wid])


def _bn1_consts(sums, p_red, g1r, b1r, n1_total):
    sum_s, sum_q, sum_cs = p_red[0:1], p_red[1:2], p_red[2:3]
    sum_c, sum_c2 = sums[0:1], sums[1:2]
    mean1 = (sum_s - _K * sum_c) / n1_total
    ey2 = (sum_q - 2.0 * sum_cs + _K * sum_c2) / n1_total
    var1 = ey2 - mean1 * mean1
    alpha = g1r[...] * lax.rsqrt(var1 + _EPS)
    beta = b1r[...] - alpha * mean1
    return alpha, beta


def _reduce_partials(p_ref):
    return jnp.sum(p_ref[...], axis=0)


def _stage1(mmn_ref, c_ref, f1t_ref, alpha, beta):
    mmn = mmn_ref[0]
    msel = jnp.where(alpha >= 0.0, mmn[:, :64], mmn[:, 64:])
    h = jnp.maximum(alpha * (msel - c_ref[0][:, :64]) + beta, 0.0)
    return jnp.concatenate([h, f1t_ref[0]], axis=1)


def _k2_body(mmn_i, c_i, f1t_i, sums_i, p_i, g1r, b1r, g_o, sz_o,
             *, n1_total):
    alpha, beta = _bn1_consts(jnp.sum(sums_i[...], axis=0),
                              _reduce_partials(p_i), g1r, b1r, n1_total)
    z = _stage1(mmn_i, c_i, f1t_i, alpha, beta)
    ztz = lax.dot_general(z, z, (((0,), (0,)), ((), ())),
                          preferred_element_type=jnp.float32, precision=_HI)
    szrow = jnp.sum(z, axis=0, keepdims=True)

    @pl.when((pl.program_id(0) == 0) & (pl.program_id(1) == 0))
    def _():
        g_o[...] = jnp.zeros_like(g_o)
        sz_o[...] = jnp.zeros_like(sz_o)

    g_o[...] += ztz
    sz_o[...] += jnp.concatenate([szrow, jnp.zeros((7, 128), jnp.float32)], 0)


def _k3_body(mmn_i, c_i, f1t_i, sums_i, p_i, g1r, b1r, g_i, sz_i, w2t,
             g2r, b2r, y_o, *, n1_total, n2_total):
    alpha, beta = _bn1_consts(jnp.sum(sums_i[...], axis=0),
                              _reduce_partials(p_i), g1r, b1r, n1_total)
    z = _stage1(mmn_i, c_i, f1t_i, alpha, beta)

    g_sum = jnp.sum(g_i[...], axis=0)
    szrow = jnp.sum(sz_i[...], axis=0)[0:1] / n2_total
    m2 = jnp.dot(szrow, w2t[...], preferred_element_type=jnp.float32,
                 precision=_HI)
    t = jnp.dot(g_sum, w2t[...], preferred_element_type=jnp.float32,
                precision=_HI)
    e2 = jnp.sum(w2t[...] * t, axis=0, keepdims=True) / n2_total
    inv2 = lax.rsqrt((e2 - m2 * m2) + _EPS)
    scale = g2r[...] * inv2
    bias = b2r[...] - m2 * scale

    y = jnp.dot(z, w2t[...], preferred_element_type=jnp.float32)
    y_o[0] = jnp.maximum(y * scale + bias, 0.0)


def kernel(pos1, pos2, feature1, feature2, W1, g1, b1, W2, g2, b2):
    B, _, N1 = pos1.shape
    N2 = pos2.shape[2]
    f32 = jnp.float32

    pos1t = jnp.transpose(pos1, (0, 2, 1))
    pos2t = jnp.transpose(pos2, (0, 2, 1))
    f2t = jnp.transpose(feature2, (0, 2, 1))
    f1t = jnp.transpose(feature1, (0, 2, 1))
    w1t = jnp.transpose(W1)
    w1f = jnp.pad(w1t[:64], ((0, 0), (0, 64)))
    w1p = jnp.pad(w1t[64:], ((0, 0), (0, 64)))
    w2t = jnp.transpose(W2)
    g1r, b1r = g1.reshape(1, 64), b1.reshape(1, 64)
    g2r, b2r = g2.reshape(1, 128), b2.reshape(1, 128)

    whole = lambda shp: pl.BlockSpec(shp, lambda b, i: (0,) * len(shp))

    a_tab = pl.pallas_call(
        _k0_body,
        grid=(B,),
        in_specs=[pl.BlockSpec((1, N2, 64), lambda b: (b, 0, 0)),
                  pl.BlockSpec((1, N2, 3), lambda b: (b, 0, 0)),
                  pl.BlockSpec((64, 128), lambda b: (0, 0)),
                  pl.BlockSpec((3, 128), lambda b: (0, 0))],
        out_specs=[pl.BlockSpec((1, N2, 128), lambda b: (b, 0, 0))],
        out_shape=[jax.ShapeDtypeStruct((B, N2, 128), f32)],
    )(f2t, pos2t, w1f, w1p)[0]

    blk = 512
    rowblk = lambda w: pl.BlockSpec((1, blk, w), lambda b, i: (b, i, 0))
    blk2 = 512
    rowblk2 = lambda w: pl.BlockSpec((1, blk2, w), lambda b, i: (b, i, 0))
    whole2 = lambda shp: pl.BlockSpec(shp, lambda b, i: (0,) * len(shp))

    info = plsc.get_sparse_core_info()
    nc, ns = info.num_cores, info.num_subcores
    nw = nc * ns
    qpw = N1 // nw

    sc = functools.partial(
        pl.kernel,
        mesh=plsc.VectorSubcoreMesh(core_axis_name="c", subcore_axis_name="s"),
        out_type=[jax.ShapeDtypeStruct((N1, 128), f32),
                  jax.ShapeDtypeStruct((nw, 16, 128), f32)],
        scratch_types=[pltpu.VMEM(((_QC * _K) // 128, 128), jnp.int32),
                       pltpu.VMEM((_QC * _K, 128), f32),
                       pltpu.VMEM((_QC, 128), f32),
                       pltpu.VMEM((_QC, 128), f32),
                       pltpu.VMEM((16, 128), f32),
                       pltpu.SemaphoreType.DMA],
    )(functools.partial(_sc_gather, qpw=qpw, nc=nc))

    k1 = pl.pallas_call(
        functools.partial(_k1_body, blk=blk, n2=N2),
        grid=(1, N1 // blk),
        in_specs=[rowblk(3), pl.BlockSpec((1, 3, N2), lambda b, i: (b, 0, 0)),
                  pl.BlockSpec((3, 128), lambda b, i: (0, 0))],
        out_specs=[rowblk(_K), rowblk(128),
                   pl.BlockSpec((8, 64), lambda b, i: (0, 0))],
        out_shape=[jax.ShapeDtypeStruct((1, N1, _K), jnp.int32),
                   jax.ShapeDtypeStruct((1, N1, 128), f32),
                   jax.ShapeDtypeStruct((8, 64), f32)],
    )

    mmn_l, c_l, sums_l, p_l = [], [], [], []
    for b in range(B):
        idx_b, c_b, sums_b = k1(pos1t[b:b + 1], pos2[b:b + 1], w1p)
        idx_3d = idx_b.reshape(N1 // _QC, (_QC * _K) // 128, 128)
        mmn_b, p_b = sc(a_tab[b], idx_3d, c_b.reshape(N1, 128))
        mmn_l.append(mmn_b)
        c_l.append(c_b)
        sums_l.append(sums_b)
        p_l.append(p_b)

    sums_all = jnp.stack(sums_l)
    p_cat = jnp.concatenate(p_l, axis=0)
    p_in = p_cat[:, :12, :16].reshape(B * nw, 3, 64)
    p_in = jnp.concatenate([p_in, jnp.zeros((B * nw, 5, 64), f32)], axis=1)

    n1_total = float(B * N1 * _K)
    n2_total = float(B * N1)

    k2 = pl.pallas_call(
        functools.partial(_k2_body, n1_total=n1_total),
        grid=(1, N1 // blk2),
        in_specs=[rowblk2(128), rowblk2(128), rowblk2(64),
                  whole2((B, 8, 64)), whole2((B * nw, 8, 64)),
                  whole2((1, 64)), whole2((1, 64))],
        out_specs=[whole2((128, 128)), whole2((8, 128))],
        out_shape=[jax.ShapeDtypeStruct((128, 128), f32),
                   jax.ShapeDtypeStruct((8, 128), f32)],
    )

    g_l, sz_l = [], []
    for b in range(B):
        g_b, sz_b = k2(mmn_l[b].reshape(1, N1, 128), c_l[b], f1t[b:b + 1],
                       sums_all, p_in, g1r, b1r)
        g_l.append(g_b)
        sz_l.append(sz_b)
    g_all = jnp.stack(g_l)
    sz_all = jnp.stack(sz_l)

    k3 = pl.pallas_call(
        functools.partial(_k3_body, n1_total=n1_total, n2_total=n2_total),
        grid=(1, N1 // blk2),
        in_specs=[rowblk2(128), rowblk2(128), rowblk2(64),
                  whole2((B, 8, 64)), whole2((B * nw, 8, 64)),
                  whole2((1, 64)), whole2((1, 64)),
                  whole2((B, 128, 128)), whole2((B, 8, 128)),
                  whole2((128, 128)), whole2((1, 128)), whole2((1, 128))],
        out_specs=[rowblk2(128)],
        out_shape=[jax.ShapeDtypeStruct((1, N1, 128), f32)],
    )

    y_l = [k3(mmn_l[b].reshape(1, N1, 128), c_l[b], f1t[b:b + 1], sums_all,
              p_in, g1r, b1r, g_all, sz_all, w2t, g2r, b2r)[0]
           for b in range(B)]
    y = jnp.concatenate(y_l, axis=0)
    return jnp.transpose(y, (0, 2, 1))


# speedup vs baseline: 1.0607x; 1.0083x over previous
import functools

import jax
import jax.numpy as jnp
from jax import lax
from jax.experimental import pallas as pl
from jax.experimental.pallas import tpu as pltpu
from jax.experimental.pallas import tpu_sc as plsc

_K = 16
_EPS = 1e-5
_QC = 32
_HI = lax.Precision.HIGHEST


def _k0_body(f2t, p2t, w1f, w1p, a_o):
    a_o[0] = (jnp.dot(f2t[0], w1f[...], preferred_element_type=jnp.float32)
              + jnp.dot(p2t[0], w1p[...], preferred_element_type=jnp.float32))


def _k1_body(p1t, p2, w1p, idx_o, c_o, sums_o, *, blk, n2):
    q = p1t[0]
    s3 = p2[0]

    cblk = jnp.dot(q, w1p[...], preferred_element_type=jnp.float32)

    qq = jnp.sum(q * q, axis=1, keepdims=True)
    ss = jnp.sum(s3 * s3, axis=0, keepdims=True)
    qs = jnp.dot(q, s3, preferred_element_type=jnp.float32)
    d = (qq - 2.0 * qs) + ss

    iota = lax.broadcasted_iota(jnp.int32, (blk, n2), 1)
    iota_k = lax.broadcasted_iota(jnp.int32, (blk, _K), 1)
    inf = jnp.float32(jnp.inf)
    boff = pl.program_id(0) * n2

    acc = jnp.zeros((blk, _K), jnp.int32)
    for t in range(_K):
        rmin = jnp.min(d, axis=1, keepdims=True)
        idx_t = jnp.min(jnp.where(d == rmin, iota, n2), axis=1, keepdims=True)
        d = jnp.where(iota == idx_t, inf, d)
        acc = jnp.where(iota_k == t, idx_t + boff, acc)

    idx_o[0] = acc
    c_o[0] = cblk
    c64 = cblk[:, :64]
    part = jnp.concatenate([
        jnp.sum(c64, axis=0, keepdims=True),
        jnp.sum(c64 * c64, axis=0, keepdims=True),
        jnp.zeros((6, 64), jnp.float32),
    ], axis=0)

    @pl.when((pl.program_id(0) == 0) & (pl.program_id(1) == 0))
    def _():
        sums_o[...] = jnp.zeros_like(sums_o)

    sums_o[...] += part


def _sc_gather(a_hbm, idx_hbm, c_hbm, mmn_hbm, p_hbm,
               idx_v, rows_v, c_v, mmn_v, p_v, sem, *, qpw, nc):
    wid = lax.axis_index("s") * nc + lax.axis_index("c")
    zero = jnp.zeros((16,), jnp.float32)
    nsub = (_QC * _K) // 128

    nchunks = qpw // _QC

    def chunk(ci, carry):
        qbase = wid * qpw + ci * _QC
        pltpu.sync_copy(idx_hbm.at[wid * nchunks + ci], idx_v)
        copies = [pltpu.async_copy(a_hbm.at[idx_v.at[i]],
                                   rows_v.at[pl.ds(i * 128, 128)], sem)
                  for i in range(nsub)]
        for cp in copies:
            cp.wait()
        pltpu.sync_copy(c_hbm.at[pl.ds(qbase, _QC)], c_v)

        def qloop(qi, pc):
            out = list(pc)
            for cb in range(4):
                sl = slice(16 * cb, 16 * (cb + 1))
                m = jnp.full((16,), -jnp.inf, jnp.float32)
                mn = jnp.full((16,), jnp.inf, jnp.float32)
                s = zero
                qsum = zero
                for j in range(_K):
                    f = rows_v[qi * _K + j, sl]
                    m = jnp.maximum(m, f)
                    mn = jnp.minimum(mn, f)
                    s = s + f
                    qsum = qsum + f * f
                mmn_v[qi, sl] = m
                mmn_v[qi, slice(64 + 16 * cb, 64 + 16 * (cb + 1))] = mn
                cv = c_v[qi, sl]
                out[cb] = out[cb] + s
                out[4 + cb] = out[4 + cb] + qsum
                out[8 + cb] = out[8 + cb] + cv * s
            return tuple(out)

        pc = lax.fori_loop(0, _QC, qloop, carry)
        pltpu.sync_copy(mmn_v, mmn_hbm.at[pl.ds(qbase, _QC)])
        return pc

    pc = lax.fori_loop(0, nchunks, chunk, tuple(zero for _ in range(12)))
    for r in range(16):
        for l in range(8):
            p_v[r, slice(16 * l, 16 * (l + 1))] = pc[r] if (r < 12 and l == 0) else zero
    pltpu.sync_copy(p_v, p_hbm.at[wid])


def _bn1_consts(sums, p_red, g1r, b1r, n1_total):
    sum_s, sum_q, sum_cs = p_red[0:1], p_red[1:2], p_red[2:3]
    sum_c, sum_c2 = sums[0:1], sums[1:2]
    mean1 = (sum_s - _K * sum_c) / n1_total
    ey2 = (sum_q - 2.0 * sum_cs + _K * sum_c2) / n1_total
    var1 = ey2 - mean1 * mean1
    alpha = g1r[...] * lax.rsqrt(var1 + _EPS)
    beta = b1r[...] - alpha * mean1
    return alpha, beta


def _reduce_partials(p_ref):
    return jnp.sum(p_ref[...], axis=0)


def _stage1(mmn_ref, c_ref, f1t_ref, alpha, beta):
    mmn = mmn_ref[0]
    msel = jnp.where(alpha >= 0.0, mmn[:, :64], mmn[:, 64:])
    h = jnp.maximum(alpha * (msel - c_ref[0][:, :64]) + beta, 0.0)
    return jnp.concatenate([h, f1t_ref[0]], axis=1)


def _k2_body(mmn_i, c_i, f1t_i, sums_i, p_i, g1r, b1r, g_o, sz_o,
             *, n1_total):
    alpha, beta = _bn1_consts(jnp.sum(sums_i[...], axis=0),
                              _reduce_partials(p_i), g1r, b1r, n1_total)
    z = _stage1(mmn_i, c_i, f1t_i, alpha, beta)
    ztz = lax.dot_general(z, z, (((0,), (0,)), ((), ())),
                          preferred_element_type=jnp.float32, precision=_HI)
    szrow = jnp.sum(z, axis=0, keepdims=True)

    @pl.when((pl.program_id(0) == 0) & (pl.program_id(1) == 0))
    def _():
        g_o[...] = jnp.zeros_like(g_o)
        sz_o[...] = jnp.zeros_like(sz_o)

    g_o[...] += ztz
    sz_o[...] += jnp.concatenate([szrow, jnp.zeros((7, 128), jnp.float32)], 0)


def _k3_body(mmn_i, c_i, f1t_i, sums_i, p_i, g1r, b1r, g_i, sz_i, w2t,
             g2r, b2r, y_o, *, n1_total, n2_total):
    alpha, beta = _bn1_consts(jnp.sum(sums_i[...], axis=0),
                              _reduce_partials(p_i), g1r, b1r, n1_total)
    z = _stage1(mmn_i, c_i, f1t_i, alpha, beta)

    g_sum = jnp.sum(g_i[...], axis=0)
    szrow = jnp.sum(sz_i[...], axis=0)[0:1] / n2_total
    m2 = jnp.dot(szrow, w2t[...], preferred_element_type=jnp.float32,
                 precision=_HI)
    t = jnp.dot(g_sum, w2t[...], preferred_element_type=jnp.float32,
                precision=_HI)
    e2 = jnp.sum(w2t[...] * t, axis=0, keepdims=True) / n2_total
    inv2 = lax.rsqrt((e2 - m2 * m2) + _EPS)
    scale = g2r[...] * inv2
    bias = b2r[...] - m2 * scale

    y = jnp.dot(z, w2t[...], preferred_element_type=jnp.float32)
    y_o[0] = jnp.transpose(jnp.maximum(y * scale + bias, 0.0))


def kernel(pos1, pos2, feature1, feature2, W1, g1, b1, W2, g2, b2):
    B, _, N1 = pos1.shape
    N2 = pos2.shape[2]
    f32 = jnp.float32

    pos1t = jnp.transpose(pos1, (0, 2, 1))
    pos2t = jnp.transpose(pos2, (0, 2, 1))
    f2t = jnp.transpose(feature2, (0, 2, 1))
    f1t = jnp.transpose(feature1, (0, 2, 1))
    w1t = jnp.transpose(W1)
    w1f = jnp.pad(w1t[:64], ((0, 0), (0, 64)))
    w1p = jnp.pad(w1t[64:], ((0, 0), (0, 64)))
    w2t = jnp.transpose(W2)
    g1r, b1r = g1.reshape(1, 64), b1.reshape(1, 64)
    g2r, b2r = g2.reshape(1, 128), b2.reshape(1, 128)

    whole = lambda shp: pl.BlockSpec(shp, lambda b, i: (0,) * len(shp))

    a_tab = pl.pallas_call(
        _k0_body,
        grid=(B,),
        in_specs=[pl.BlockSpec((1, N2, 64), lambda b: (b, 0, 0)),
                  pl.BlockSpec((1, N2, 3), lambda b: (b, 0, 0)),
                  pl.BlockSpec((64, 128), lambda b: (0, 0)),
                  pl.BlockSpec((3, 128), lambda b: (0, 0))],
        out_specs=[pl.BlockSpec((1, N2, 128), lambda b: (b, 0, 0))],
        out_shape=[jax.ShapeDtypeStruct((B, N2, 128), f32)],
    )(f2t, pos2t, w1f, w1p)[0]

    blk = 512
    rowblk = lambda w: pl.BlockSpec((1, blk, w), lambda b, i: (b, i, 0))
    blk2 = 512
    rowblk2 = lambda w: pl.BlockSpec((1, blk2, w), lambda b, i: (b, i, 0))
    whole2 = lambda shp: pl.BlockSpec(shp, lambda b, i: (0,) * len(shp))

    info = plsc.get_sparse_core_info()
    nc, ns = info.num_cores, info.num_subcores
    nw = nc * ns
    qpw = N1 // nw

    sc = functools.partial(
        pl.kernel,
        mesh=plsc.VectorSubcoreMesh(core_axis_name="c", subcore_axis_name="s"),
        out_type=[jax.ShapeDtypeStruct((N1, 128), f32),
                  jax.ShapeDtypeStruct((nw, 16, 128), f32)],
        scratch_types=[pltpu.VMEM(((_QC * _K) // 128, 128), jnp.int32),
                       pltpu.VMEM((_QC * _K, 128), f32),
                       pltpu.VMEM((_QC, 128), f32),
                       pltpu.VMEM((_QC, 128), f32),
                       pltpu.VMEM((16, 128), f32),
                       pltpu.SemaphoreType.DMA],
    )(functools.partial(_sc_gather, qpw=qpw, nc=nc))

    k1 = pl.pallas_call(
        functools.partial(_k1_body, blk=blk, n2=N2),
        grid=(1, N1 // blk),
        in_specs=[rowblk(3), pl.BlockSpec((1, 3, N2), lambda b, i: (b, 0, 0)),
                  pl.BlockSpec((3, 128), lambda b, i: (0, 0))],
        out_specs=[rowblk(_K), rowblk(128),
                   pl.BlockSpec((8, 64), lambda b, i: (0, 0))],
        out_shape=[jax.ShapeDtypeStruct((1, N1, _K), jnp.int32),
                   jax.ShapeDtypeStruct((1, N1, 128), f32),
                   jax.ShapeDtypeStruct((8, 64), f32)],
    )

    mmn_l, c_l, sums_l, p_l = [], [], [], []
    for b in range(B):
        idx_b, c_b, sums_b = k1(pos1t[b:b + 1], pos2[b:b + 1], w1p)
        idx_3d = idx_b.reshape(N1 // _QC, (_QC * _K) // 128, 128)
        mmn_b, p_b = sc(a_tab[b], idx_3d, c_b.reshape(N1, 128))
        mmn_l.append(mmn_b)
        c_l.append(c_b)
        sums_l.append(sums_b)
        p_l.append(p_b)

    sums_all = jnp.stack(sums_l)
    p_cat = jnp.concatenate(p_l, axis=0)
    p_in = p_cat[:, :12, :16].reshape(B * nw, 3, 64)
    p_in = jnp.concatenate([p_in, jnp.zeros((B * nw, 5, 64), f32)], axis=1)

    n1_total = float(B * N1 * _K)
    n2_total = float(B * N1)

    k2 = pl.pallas_call(
        functools.partial(_k2_body, n1_total=n1_total),
        grid=(1, N1 // blk2),
        in_specs=[rowblk2(128), rowblk2(128), rowblk2(64),
                  whole2((B, 8, 64)), whole2((B * nw, 8, 64)),
                  whole2((1, 64)), whole2((1, 64))],
        out_specs=[whole2((128, 128)), whole2((8, 128))],
        out_shape=[jax.ShapeDtypeStruct((128, 128), f32),
                   jax.ShapeDtypeStruct((8, 128), f32)],
    )

    g_l, sz_l = [], []
    for b in range(B):
        g_b, sz_b = k2(mmn_l[b].reshape(1, N1, 128), c_l[b], f1t[b:b + 1],
                       sums_all, p_in, g1r, b1r)
        g_l.append(g_b)
        sz_l.append(sz_b)
    g_all = jnp.stack(g_l)
    sz_all = jnp.stack(sz_l)

    k3 = pl.pallas_call(
        functools.partial(_k3_body, n1_total=n1_total, n2_total=n2_total),
        grid=(1, N1 // blk2),
        in_specs=[rowblk2(128), rowblk2(128), rowblk2(64),
                  whole2((B, 8, 64)), whole2((B * nw, 8, 64)),
                  whole2((1, 64)), whole2((1, 64)),
                  whole2((B, 128, 128)), whole2((B, 8, 128)),
                  whole2((128, 128)), whole2((1, 128)), whole2((1, 128))],
        out_specs=[pl.BlockSpec((1, 128, blk2), lambda b, i: (b, 0, i))],
        out_shape=[jax.ShapeDtypeStruct((1, 128, N1), f32)],
    )

    y_l = [k3(mmn_l[b].reshape(1, N1, 128), c_l[b], f1t[b:b + 1], sums_all,
              p_in, g1r, b1r, g_all, sz_all, w2t, g2r, b2r)[0]
           for b in range(B)]
    return jnp.concatenate(y_l, axis=0)


# speedup vs baseline: 1.2063x; 1.1373x over previous
import functools

import jax
import jax.numpy as jnp
from jax import lax
from jax.experimental import pallas as pl
from jax.experimental.pallas import tpu as pltpu
from jax.experimental.pallas import tpu_sc as plsc

_K = 16
_EPS = 1e-5
_QC = 32
_HI = lax.Precision.HIGHEST


def _k0_body(f2t, p2t, w1f, w1p, a_o):
    a_o[0] = (jnp.dot(f2t[0], w1f[...], preferred_element_type=jnp.float32)
              + jnp.dot(p2t[0], w1p[...], preferred_element_type=jnp.float32))


def _k1_body(p1t, p2, w1p, idx_o, c_o, sums_o, *, blk, n2):
    q = p1t[0]
    s3 = p2[0]

    cblk = jnp.dot(q, w1p[...], preferred_element_type=jnp.float32)

    qq = jnp.sum(q * q, axis=1, keepdims=True)
    ss = jnp.sum(s3 * s3, axis=0, keepdims=True)
    qs = jnp.dot(q, s3, preferred_element_type=jnp.float32)
    d = (qq - 2.0 * qs) + ss

    iota = lax.broadcasted_iota(jnp.int32, (blk, n2), 1)
    iota_k = lax.broadcasted_iota(jnp.int32, (blk, _K), 1)
    inf = jnp.float32(jnp.inf)
    boff = pl.program_id(0) * n2

    acc = jnp.zeros((blk, _K), jnp.int32)
    for t in range(_K):
        idx_t = jnp.argmin(d, axis=1).astype(jnp.int32)[:, None]
        d = jnp.where(iota == idx_t, inf, d)
        acc = jnp.where(iota_k == t, idx_t + boff, acc)

    idx_o[0] = acc
    c_o[0] = cblk
    c64 = cblk[:, :64]
    part = jnp.concatenate([
        jnp.sum(c64, axis=0, keepdims=True),
        jnp.sum(c64 * c64, axis=0, keepdims=True),
        jnp.zeros((6, 64), jnp.float32),
    ], axis=0)

    @pl.when((pl.program_id(0) == 0) & (pl.program_id(1) == 0))
    def _():
        sums_o[...] = jnp.zeros_like(sums_o)

    sums_o[...] += part


def _sc_gather(a_hbm, idx_hbm, c_hbm, mmn_hbm, p_hbm,
               idx_v, rows_v, c_v, mmn_v, p_v, sem, *, qpw, nc):
    wid = lax.axis_index("s") * nc + lax.axis_index("c")
    zero = jnp.zeros((16,), jnp.float32)
    nsub = (_QC * _K) // 128

    nchunks = qpw // _QC

    def chunk(ci, carry):
        qbase = wid * qpw + ci * _QC
        pltpu.sync_copy(idx_hbm.at[wid * nchunks + ci], idx_v)
        copies = [pltpu.async_copy(a_hbm.at[idx_v.at[i]],
                                   rows_v.at[pl.ds(i * 128, 128)], sem)
                  for i in range(nsub)]
        for cp in copies:
            cp.wait()
        pltpu.sync_copy(c_hbm.at[pl.ds(qbase, _QC)], c_v)

        def qloop(qi, pc):
            out = list(pc)
            for cb in range(4):
                sl = slice(16 * cb, 16 * (cb + 1))
                m = jnp.full((16,), -jnp.inf, jnp.float32)
                mn = jnp.full((16,), jnp.inf, jnp.float32)
                s = zero
                qsum = zero
                for j in range(_K):
                    f = rows_v[qi * _K + j, sl]
                    m = jnp.maximum(m, f)
                    mn = jnp.minimum(mn, f)
                    s = s + f
                    qsum = qsum + f * f
                mmn_v[qi, sl] = m
                mmn_v[qi, slice(64 + 16 * cb, 64 + 16 * (cb + 1))] = mn
                cv = c_v[qi, sl]
                out[cb] = out[cb] + s
                out[4 + cb] = out[4 + cb] + qsum
                out[8 + cb] = out[8 + cb] + cv * s
            return tuple(out)

        pc = lax.fori_loop(0, _QC, qloop, carry)
        pltpu.sync_copy(mmn_v, mmn_hbm.at[pl.ds(qbase, _QC)])
        return pc

    pc = lax.fori_loop(0, nchunks, chunk, tuple(zero for _ in range(12)))
    for r in range(16):
        for l in range(8):
            p_v[r, slice(16 * l, 16 * (l + 1))] = pc[r] if (r < 12 and l == 0) else zero
    pltpu.sync_copy(p_v, p_hbm.at[wid])


def _bn1_consts(sums, p_red, g1r, b1r, n1_total):
    sum_s, sum_q, sum_cs = p_red[0:1], p_red[1:2], p_red[2:3]
    sum_c, sum_c2 = sums[0:1], sums[1:2]
    mean1 = (sum_s - _K * sum_c) / n1_total
    ey2 = (sum_q - 2.0 * sum_cs + _K * sum_c2) / n1_total
    var1 = ey2 - mean1 * mean1
    alpha = g1r[...] * lax.rsqrt(var1 + _EPS)
    beta = b1r[...] - alpha * mean1
    return alpha, beta


def _reduce_partials(p_ref):
    return jnp.sum(p_ref[...], axis=0)


def _stage1(mmn_ref, c_ref, f1t_ref, alpha, beta):
    mmn = mmn_ref[0]
    msel = jnp.where(alpha >= 0.0, mmn[:, :64], mmn[:, 64:])
    h = jnp.maximum(alpha * (msel - c_ref[0][:, :64]) + beta, 0.0)
    return jnp.concatenate([h, f1t_ref[0]], axis=1)


def _k2_body(mmn_i, c_i, f1t_i, sums_i, p_i, g1r, b1r, g_o, sz_o,
             *, n1_total):
    alpha, beta = _bn1_consts(jnp.sum(sums_i[...], axis=0),
                              _reduce_partials(p_i), g1r, b1r, n1_total)
    z = _stage1(mmn_i, c_i, f1t_i, alpha, beta)
    ztz = lax.dot_general(z, z, (((0,), (0,)), ((), ())),
                          preferred_element_type=jnp.float32, precision=_HI)
    szrow = jnp.sum(z, axis=0, keepdims=True)

    @pl.when((pl.program_id(0) == 0) & (pl.program_id(1) == 0))
    def _():
        g_o[...] = jnp.zeros_like(g_o)
        sz_o[...] = jnp.zeros_like(sz_o)

    g_o[...] += ztz
    sz_o[...] += jnp.concatenate([szrow, jnp.zeros((7, 128), jnp.float32)], 0)


def _k3_body(mmn_i, c_i, f1t_i, sums_i, p_i, g1r, b1r, g_i, sz_i, w2t,
             g2r, b2r, y_o, *, n1_total, n2_total):
    alpha, beta = _bn1_consts(jnp.sum(sums_i[...], axis=0),
                              _reduce_partials(p_i), g1r, b1r, n1_total)
    z = _stage1(mmn_i, c_i, f1t_i, alpha, beta)

    g_sum = jnp.sum(g_i[...], axis=0)
    szrow = jnp.sum(sz_i[...], axis=0)[0:1] / n2_total
    m2 = jnp.dot(szrow, w2t[...], preferred_element_type=jnp.float32,
                 precision=_HI)
    t = jnp.dot(g_sum, w2t[...], preferred_element_type=jnp.float32,
                precision=_HI)
    e2 = jnp.sum(w2t[...] * t, axis=0, keepdims=True) / n2_total
    inv2 = lax.rsqrt((e2 - m2 * m2) + _EPS)
    scale = g2r[...] * inv2
    bias = b2r[...] - m2 * scale

    y = jnp.dot(z, w2t[...], preferred_element_type=jnp.float32)
    y_o[0] = jnp.transpose(jnp.maximum(y * scale + bias, 0.0))


def kernel(pos1, pos2, feature1, feature2, W1, g1, b1, W2, g2, b2):
    B, _, N1 = pos1.shape
    N2 = pos2.shape[2]
    f32 = jnp.float32

    pos1t = jnp.transpose(pos1, (0, 2, 1))
    pos2t = jnp.transpose(pos2, (0, 2, 1))
    f2t = jnp.transpose(feature2, (0, 2, 1))
    f1t = jnp.transpose(feature1, (0, 2, 1))
    w1t = jnp.transpose(W1)
    w1f = jnp.pad(w1t[:64], ((0, 0), (0, 64)))
    w1p = jnp.pad(w1t[64:], ((0, 0), (0, 64)))
    w2t = jnp.transpose(W2)
    g1r, b1r = g1.reshape(1, 64), b1.reshape(1, 64)
    g2r, b2r = g2.reshape(1, 128), b2.reshape(1, 128)

    whole = lambda shp: pl.BlockSpec(shp, lambda b, i: (0,) * len(shp))

    a_tab = pl.pallas_call(
        _k0_body,
        grid=(B,),
        in_specs=[pl.BlockSpec((1, N2, 64), lambda b: (b, 0, 0)),
                  pl.BlockSpec((1, N2, 3), lambda b: (b, 0, 0)),
                  pl.BlockSpec((64, 128), lambda b: (0, 0)),
                  pl.BlockSpec((3, 128), lambda b: (0, 0))],
        out_specs=[pl.BlockSpec((1, N2, 128), lambda b: (b, 0, 0))],
        out_shape=[jax.ShapeDtypeStruct((B, N2, 128), f32)],
    )(f2t, pos2t, w1f, w1p)[0]

    blk = 512
    rowblk = lambda w: pl.BlockSpec((1, blk, w), lambda b, i: (b, i, 0))
    blk2 = 512
    rowblk2 = lambda w: pl.BlockSpec((1, blk2, w), lambda b, i: (b, i, 0))
    whole2 = lambda shp: pl.BlockSpec(shp, lambda b, i: (0,) * len(shp))

    info = plsc.get_sparse_core_info()
    nc, ns = info.num_cores, info.num_subcores
    nw = nc * ns
    qpw = N1 // nw

    sc = functools.partial(
        pl.kernel,
        mesh=plsc.VectorSubcoreMesh(core_axis_name="c", subcore_axis_name="s"),
        out_type=[jax.ShapeDtypeStruct((N1, 128), f32),
                  jax.ShapeDtypeStruct((nw, 16, 128), f32)],
        scratch_types=[pltpu.VMEM(((_QC * _K) // 128, 128), jnp.int32),
                       pltpu.VMEM((_QC * _K, 128), f32),
                       pltpu.VMEM((_QC, 128), f32),
                       pltpu.VMEM((_QC, 128), f32),
                       pltpu.VMEM((16, 128), f32),
                       pltpu.SemaphoreType.DMA],
    )(functools.partial(_sc_gather, qpw=qpw, nc=nc))

    k1 = pl.pallas_call(
        functools.partial(_k1_body, blk=blk, n2=N2),
        grid=(1, N1 // blk),
        in_specs=[rowblk(3), pl.BlockSpec((1, 3, N2), lambda b, i: (b, 0, 0)),
                  pl.BlockSpec((3, 128), lambda b, i: (0, 0))],
        out_specs=[rowblk(_K), rowblk(128),
                   pl.BlockSpec((8, 64), lambda b, i: (0, 0))],
        out_shape=[jax.ShapeDtypeStruct((1, N1, _K), jnp.int32),
                   jax.ShapeDtypeStruct((1, N1, 128), f32),
                   jax.ShapeDtypeStruct((8, 64), f32)],
    )

    mmn_l, c_l, sums_l, p_l = [], [], [], []
    for b in range(B):
        idx_b, c_b, sums_b = k1(pos1t[b:b + 1], pos2[b:b + 1], w1p)
        idx_3d = idx_b.reshape(N1 // _QC, (_QC * _K) // 128, 128)
        mmn_b, p_b = sc(a_tab[b], idx_3d, c_b.reshape(N1, 128))
        mmn_l.append(mmn_b)
        c_l.append(c_b)
        sums_l.append(sums_b)
        p_l.append(p_b)

    sums_all = jnp.stack(sums_l)
    p_cat = jnp.concatenate(p_l, axis=0)
    p_in = p_cat[:, :12, :16].reshape(B * nw, 3, 64)
    p_in = jnp.concatenate([p_in, jnp.zeros((B * nw, 5, 64), f32)], axis=1)

    n1_total = float(B * N1 * _K)
    n2_total = float(B * N1)

    k2 = pl.pallas_call(
        functools.partial(_k2_body, n1_total=n1_total),
        grid=(1, N1 // blk2),
        in_specs=[rowblk2(128), rowblk2(128), rowblk2(64),
                  whole2((B, 8, 64)), whole2((B * nw, 8, 64)),
                  whole2((1, 64)), whole2((1, 64))],
        out_specs=[whole2((128, 128)), whole2((8, 128))],
        out_shape=[jax.ShapeDtypeStruct((128, 128), f32),
                   jax.ShapeDtypeStruct((8, 128), f32)],
    )

    g_l, sz_l = [], []
    for b in range(B):
        g_b, sz_b = k2(mmn_l[b].reshape(1, N1, 128), c_l[b], f1t[b:b + 1],
                       sums_all, p_in, g1r, b1r)
        g_l.append(g_b)
        sz_l.append(sz_b)
    g_all = jnp.stack(g_l)
    sz_all = jnp.stack(sz_l)

    k3 = pl.pallas_call(
        functools.partial(_k3_body, n1_total=n1_total, n2_total=n2_total),
        grid=(1, N1 // blk2),
        in_specs=[rowblk2(128), rowblk2(128), rowblk2(64),
                  whole2((B, 8, 64)), whole2((B * nw, 8, 64)),
                  whole2((1, 64)), whole2((1, 64)),
                  whole2((B, 128, 128)), whole2((B, 8, 128)),
                  whole2((128, 128)), whole2((1, 128)), whole2((1, 128))],
        out_specs=[pl.BlockSpec((1, 128, blk2), lambda b, i: (b, 0, i))],
        out_shape=[jax.ShapeDtypeStruct((1, 128, N1), f32)],
    )

    y_l = [k3(mmn_l[b].reshape(1, N1, 128), c_l[b], f1t[b:b + 1], sums_all,
              p_in, g1r, b1r, g_all, sz_all, w2t, g2r, b2r)[0]
           for b in range(B)]
    return jnp.concatenate(y_l, axis=0)
